# baseline (speedup 1.0000x reference)
import functools

import jax
import jax.numpy as jnp
from jax import lax
from jax.experimental import pallas as pl
from jax.experimental.pallas import tpu as pltpu
from jax.experimental.pallas import tpu_sc as plsc

N = 10000
E = 320000
D = 128

NC = 2
NS = 16
NW = NC * NS

C = 128
CHUNKS = E // C
FULL = CHUNKS // NW
REM = CHUNKS % NW

ACC_ROWS = 10112
ROWS_PER_TILE = ACC_ROWS // NS
DUMMY = N


def _sc_body(ei_hbm, x_hbm, out_hbm, row_v0, col_v0, row_v1,
             col_v1, row_v2, col_v2, buf0, buf1, buf2, acc,
             gsem0, gsem1, gsem2, ssem0, ssem1, ssem2,
             isem0, isem1, isem2):
    c = lax.axis_index("c")
    s = lax.axis_index("s")
    wid = c * NS + s

    def _zero_row(r, carry):
        for j in range(D // 16):
            buf0[r, pl.ds(j * 16, 16)] = jnp.zeros((16,), jnp.float32)
        return carry

    lax.fori_loop(0, C, _zero_row, 0, unroll=False)
    for b in range(ROWS_PER_TILE // C):
        pltpu.sync_copy(buf0, acc.at[pl.ds(s * ROWS_PER_TILE + b * C, C)])
    rem_rows = ROWS_PER_TILE % C
    pltpu.sync_copy(
        buf0.at[pl.ds(0, rem_rows)],
        acc.at[pl.ds(s * ROWS_PER_TILE + (ROWS_PER_TILE // C) * C,
                     rem_rows)])
    plsc.subcore_barrier()

    def _fetch_idx_start(k, row_v, col_v, sem):
        base = k * C
        r = pltpu.async_copy(ei_hbm.at[0, pl.ds(base, C)], row_v, sem)
        q = pltpu.async_copy(ei_hbm.at[1, pl.ds(base, C)], col_v, sem)
        return (r, q)

    def _fetch_idx_finish(descs, row_v, col_v):
        descs[0].wait()
        descs[1].wait()
        for j in range(C // 16):
            rv = row_v[pl.ds(j * 16, 16)]
            cv = col_v[pl.ds(j * 16, 16)]
            row_v[pl.ds(j * 16, 16)] = jnp.where(rv == cv, DUMMY, rv)

    def _fetch_idx(k, row_v, col_v):
        _fetch_idx_finish(_fetch_idx_start(k, row_v, col_v, gsem0),
                          row_v, col_v)

    def _sextet(k6, wid):
        ka = k6 * 6 * NW + wid
        kb, kc, kd, ke, kf = (ka + NW, ka + 2 * NW, ka + 3 * NW,
                              ka + 4 * NW, ka + 5 * NW)
        ia = _fetch_idx_start(ka, row_v0, col_v0, isem0)
        ib = _fetch_idx_start(kb, row_v1, col_v1, isem1)
        ic = _fetch_idx_start(kc, row_v2, col_v2, isem2)
        _fetch_idx_finish(ia, row_v0, col_v0)
        ga = pltpu.async_copy(x_hbm.at[col_v0], buf0, gsem0)
        _fetch_idx_finish(ib, row_v1, col_v1)
        ga.wait()
        sa = pltpu.async_copy(buf0, acc.at[row_v0], ssem0, add=True)
        gb = pltpu.async_copy(x_hbm.at[col_v1], buf1, gsem1)
        _fetch_idx_finish(ic, row_v2, col_v2)
        gb.wait()
        sb = pltpu.async_copy(buf1, acc.at[row_v1], ssem1, add=True)
        gc = pltpu.async_copy(x_hbm.at[col_v2], buf2, gsem2)
        sa.wait()
        i_d = _fetch_idx_start(kd, row_v0, col_v0, isem0)
        gc.wait()
        sc = pltpu.async_copy(buf2, acc.at[row_v2], ssem2, add=True)
        _fetch_idx_finish(i_d, row_v0, col_v0)
        gd = pltpu.async_copy(x_hbm.at[col_v0], buf0, gsem0)
        sb.wait()
        ie = _fetch_idx_start(ke, row_v1, col_v1, isem1)
        gd.wait()
        sd = pltpu.async_copy(buf0, acc.at[row_v0], ssem0, add=True)
        _fetch_idx_finish(ie, row_v1, col_v1)
        ge = pltpu.async_copy(x_hbm.at[col_v1], buf1, gsem1)
        sc.wait()
        i_f = _fetch_idx_start(kf, row_v2, col_v2, isem2)
        ge.wait()
        se = pltpu.async_copy(buf1, acc.at[row_v1], ssem1, add=True)
        _fetch_idx_finish(i_f, row_v2, col_v2)
        gf = pltpu.async_copy(x_hbm.at[col_v2], buf2, gsem2)
        gf.wait()
        sf = pltpu.async_copy(buf2, acc.at[row_v2], ssem2, add=True)
        sd.wait()
        se.wait()
        sf.wait()
        return wid

    lax.fori_loop(0, FULL // 6, _sextet, wid, unroll=False)

    @pl.when(wid < REM)
    def _tail():
        _fetch_idx(FULL * NW + wid, row_v0, col_v0)
        pltpu.async_copy(x_hbm.at[col_v0], buf0, gsem0).wait()
        pltpu.sync_copy(buf0, acc.at[row_v0], add=True)

    plsc.subcore_barrier()

    wds = []
    for b in range(ROWS_PER_TILE // C):
        off = s * ROWS_PER_TILE + b * C
        wds.append(pltpu.async_copy(acc.at[pl.ds(off, C)],
                                    out_hbm.at[c, pl.ds(off, C)], gsem0))
    for wd in wds:
        wd.wait()
    off = s * ROWS_PER_TILE + (ROWS_PER_TILE // C) * C
    pltpu.sync_copy(acc.at[pl.ds(off, rem_rows)],
                    out_hbm.at[c, pl.ds(off, rem_rows)])


_sc_aggregate = functools.partial(
    pl.kernel,
    mesh=plsc.VectorSubcoreMesh(core_axis_name="c", subcore_axis_name="s"),
    out_type=jax.ShapeDtypeStruct((NC, ACC_ROWS, D), jnp.float32),
    scratch_types=[
        pltpu.VMEM((C,), jnp.int32),
        pltpu.VMEM((C,), jnp.int32),
        pltpu.VMEM((C,), jnp.int32),
        pltpu.VMEM((C,), jnp.int32),
        pltpu.VMEM((C,), jnp.int32),
        pltpu.VMEM((C,), jnp.int32),
        pltpu.VMEM((C, D), jnp.float32),
        pltpu.VMEM((C, D), jnp.float32),
        pltpu.VMEM((C, D), jnp.float32),
        pltpu.VMEM_SHARED((ACC_ROWS, D), jnp.float32),
    ] + [pltpu.SemaphoreType.DMA] * 9,
)(_sc_body)


def _mlp_body(x_ref, p_ref, w1_ref, b1_ref, w2_ref, b2_ref, o_ref):
    out = x_ref[...] + p_ref[0] + p_ref[1]
    h = jnp.dot(out, w1_ref[...], preferred_element_type=jnp.float32)
    h = jnp.maximum(h + b1_ref[...], 0.0)
    y = jnp.dot(h, w2_ref[...], preferred_element_type=jnp.float32)
    o_ref[...] = y + b2_ref[...]


MB = 1000


def _mlp(x, partials, W1, b1, W2, b2):
    grid = (N // MB,)
    return pl.pallas_call(
        _mlp_body,
        grid=grid,
        in_specs=[
            pl.BlockSpec((MB, D), lambda i: (i, 0)),
            pl.BlockSpec((NC, MB, D), lambda i: (0, i, 0)),
            pl.BlockSpec((D, D), lambda i: (0, 0)),
            pl.BlockSpec((1, D), lambda i: (0, 0)),
            pl.BlockSpec((D, D), lambda i: (0, 0)),
            pl.BlockSpec((1, D), lambda i: (0, 0)),
        ],
        out_specs=pl.BlockSpec((MB, D), lambda i: (i, 0)),
        out_shape=jax.ShapeDtypeStruct((N, D), jnp.float32),
    )(x, partials, W1, b1.reshape(1, D), W2, b2.reshape(1, D))


def kernel(x, edge_index, W1, b1, W2, b2):
    ei = edge_index.astype(jnp.int32)
    partials = _sc_aggregate(ei, x)
    return _mlp(x, partials, W1, b1, W2, b2)


# speedup vs baseline: 1.0522x; 1.0522x over previous
import functools

import jax
import jax.numpy as jnp
from jax import lax
from jax.experimental import pallas as pl
from jax.experimental.pallas import tpu as pltpu
from jax.experimental.pallas import tpu_sc as plsc

N = 10000
E = 320000
D = 128

NC = 2
NS = 16
NW = NC * NS

C = 128
CHUNKS = E // C
FULL = CHUNKS // NW
REM = CHUNKS % NW

ACC_ROWS = 10112
ROWS_PER_TILE = ACC_ROWS // NS
DUMMY = N


def _sc_body(ei_hbm, x_hbm, out_hbm, row_v0, col_v0, row_v1,
             col_v1, row_v2, col_v2, buf0, buf1, buf2, acc,
             gsem0, gsem1, gsem2, ssem0, ssem1, ssem2,
             isem0, isem1, isem2):
    c = lax.axis_index("c")
    s = lax.axis_index("s")
    wid = c * NS + s

    def _zero_row(r, carry):
        for j in range(D // 16):
            buf0[r, pl.ds(j * 16, 16)] = jnp.zeros((16,), jnp.float32)
        return carry

    lax.fori_loop(0, C, _zero_row, 0, unroll=False)
    for b in range(ROWS_PER_TILE // C):
        pltpu.sync_copy(buf0, acc.at[pl.ds(s * ROWS_PER_TILE + b * C, C)])
    rem_rows = ROWS_PER_TILE % C
    pltpu.sync_copy(
        buf0.at[pl.ds(0, rem_rows)],
        acc.at[pl.ds(s * ROWS_PER_TILE + (ROWS_PER_TILE // C) * C,
                     rem_rows)])
    plsc.subcore_barrier()

    def _fetch_idx_start(k, row_v, col_v, sem):
        base = k * C
        r = pltpu.async_copy(ei_hbm.at[0, pl.ds(base, C)], row_v, sem)
        q = pltpu.async_copy(ei_hbm.at[1, pl.ds(base, C)], col_v, sem)
        return (r, q)

    def _fetch_idx_finish(descs, row_v, col_v):
        descs[0].wait()
        descs[1].wait()
        for j in range(C // 16):
            rv = row_v[pl.ds(j * 16, 16)]
            cv = col_v[pl.ds(j * 16, 16)]
            row_v[pl.ds(j * 16, 16)] = jnp.where(rv == cv, DUMMY, rv)

    def _fetch_idx(k, row_v, col_v):
        _fetch_idx_finish(_fetch_idx_start(k, row_v, col_v, gsem0),
                          row_v, col_v)

    rvs = (row_v0, row_v1, row_v2)
    cvs = (col_v0, col_v1, col_v2)
    bufs = (buf0, buf1, buf2)
    isems = (isem0, isem1, isem2)
    gsems = (gsem0, gsem1, gsem2)
    ssems = (ssem0, ssem1, ssem2)

    def _block(chunk_of, n):
        idx_d = [None] * n
        g = [None] * n
        sct = [None] * n

        def idx_start(i):
            idx_d[i] = _fetch_idx_start(chunk_of(i), rvs[i % 3], cvs[i % 3],
                                        isems[i % 3])

        def idx_finish(i):
            _fetch_idx_finish(idx_d[i], rvs[i % 3], cvs[i % 3])

        def gather(i):
            g[i] = pltpu.async_copy(x_hbm.at[cvs[i % 3]], bufs[i % 3],
                                    gsems[i % 3])

        def scatter(i):
            sct[i] = pltpu.async_copy(bufs[i % 3], acc.at[rvs[i % 3]],
                                      ssems[i % 3], add=True)

        idx_start(0)
        idx_start(1)
        idx_start(2)
        idx_finish(0)
        gather(0)
        idx_finish(1)
        g[0].wait()
        scatter(0)
        gather(1)
        idx_finish(2)
        g[1].wait()
        scatter(1)
        gather(2)
        for i in range(3, n):
            sct[i - 3].wait()
            idx_start(i)
            g[i - 1].wait()
            scatter(i - 1)
            idx_finish(i)
            gather(i)
        g[n - 1].wait()
        scatter(n - 1)
        for i in range(n - 3, n):
            sct[i].wait()

    BN = 12

    def _block12(kb, wid):
        _block(lambda i: (kb * BN + i) * NW + wid, BN)
        return wid

    lax.fori_loop(0, FULL // BN, _block12, wid, unroll=False)
    _block(lambda i: ((FULL // BN) * BN + i) * NW + wid, FULL % BN)

    @pl.when(wid < REM)
    def _tail():
        _fetch_idx(FULL * NW + wid, row_v0, col_v0)
        pltpu.async_copy(x_hbm.at[col_v0], buf0, gsem0).wait()
        pltpu.sync_copy(buf0, acc.at[row_v0], add=True)

    plsc.subcore_barrier()

    wds = []
    for b in range(ROWS_PER_TILE // C):
        off = s * ROWS_PER_TILE + b * C
        wds.append(pltpu.async_copy(acc.at[pl.ds(off, C)],
                                    out_hbm.at[c, pl.ds(off, C)], gsem0))
    for wd in wds:
        wd.wait()
    off = s * ROWS_PER_TILE + (ROWS_PER_TILE // C) * C
    pltpu.sync_copy(acc.at[pl.ds(off, rem_rows)],
                    out_hbm.at[c, pl.ds(off, rem_rows)])


_sc_aggregate = functools.partial(
    pl.kernel,
    mesh=plsc.VectorSubcoreMesh(core_axis_name="c", subcore_axis_name="s"),
    out_type=jax.ShapeDtypeStruct((NC, ACC_ROWS, D), jnp.float32),
    scratch_types=[
        pltpu.VMEM((C,), jnp.int32),
        pltpu.VMEM((C,), jnp.int32),
        pltpu.VMEM((C,), jnp.int32),
        pltpu.VMEM((C,), jnp.int32),
        pltpu.VMEM((C,), jnp.int32),
        pltpu.VMEM((C,), jnp.int32),
        pltpu.VMEM((C, D), jnp.float32),
        pltpu.VMEM((C, D), jnp.float32),
        pltpu.VMEM((C, D), jnp.float32),
        pltpu.VMEM_SHARED((ACC_ROWS, D), jnp.float32),
    ] + [pltpu.SemaphoreType.DMA] * 9,
)(_sc_body)


def _mlp_body(x_ref, p_ref, w1_ref, b1_ref, w2_ref, b2_ref, o_ref):
    out = x_ref[...] + p_ref[0] + p_ref[1]
    h = jnp.dot(out, w1_ref[...], preferred_element_type=jnp.float32)
    h = jnp.maximum(h + b1_ref[...], 0.0)
    y = jnp.dot(h, w2_ref[...], preferred_element_type=jnp.float32)
    o_ref[...] = y + b2_ref[...]


MB = 2000


def _mlp(x, partials, W1, b1, W2, b2):
    grid = (N // MB,)
    return pl.pallas_call(
        _mlp_body,
        grid=grid,
        in_specs=[
            pl.BlockSpec((MB, D), lambda i: (i, 0)),
            pl.BlockSpec((NC, MB, D), lambda i: (0, i, 0)),
            pl.BlockSpec((D, D), lambda i: (0, 0)),
            pl.BlockSpec((1, D), lambda i: (0, 0)),
            pl.BlockSpec((D, D), lambda i: (0, 0)),
            pl.BlockSpec((1, D), lambda i: (0, 0)),
        ],
        out_specs=pl.BlockSpec((MB, D), lambda i: (i, 0)),
        out_shape=jax.ShapeDtypeStruct((N, D), jnp.float32),
    )(x, partials, W1, b1.reshape(1, D), W2, b2.reshape(1, D))


def kernel(x, edge_index, W1, b1, W2, b2):
    ei = edge_index.astype(jnp.int32)
    partials = _sc_aggregate(ei, x)
    return _mlp(x, partials, W1, b1, W2, b2)


# speedup vs baseline: 1.0639x; 1.0112x over previous
import functools

import jax
import jax.numpy as jnp
from jax import lax
from jax.experimental import pallas as pl
from jax.experimental.pallas import tpu as pltpu
from jax.experimental.pallas import tpu_sc as plsc

N = 10000
E = 320000
D = 128

NC = 2
NS = 16
NW = NC * NS

C = 128
CHUNKS = E // C
FULL = CHUNKS // NW
REM = CHUNKS % NW

ACC_ROWS = 10112
ROWS_PER_TILE = ACC_ROWS // NS
DUMMY = N


def _sc_body(ei_hbm, x_hbm, out_hbm, row_v0, col_v0, row_v1,
             col_v1, row_v2, col_v2, buf0, buf1, buf2, acc,
             gsem0, gsem1, gsem2, ssem0, ssem1, ssem2,
             isem0, isem1, isem2):
    c = lax.axis_index("c")
    s = lax.axis_index("s")
    wid = c * NS + s

    def _zero_row(r, carry):
        for j in range(D // 16):
            buf0[r, pl.ds(j * 16, 16)] = jnp.zeros((16,), jnp.float32)
        return carry

    lax.fori_loop(0, C, _zero_row, 0, unroll=False)
    for b in range(ROWS_PER_TILE // C):
        pltpu.sync_copy(buf0, acc.at[pl.ds(s * ROWS_PER_TILE + b * C, C)])
    rem_rows = ROWS_PER_TILE % C
    pltpu.sync_copy(
        buf0.at[pl.ds(0, rem_rows)],
        acc.at[pl.ds(s * ROWS_PER_TILE + (ROWS_PER_TILE // C) * C,
                     rem_rows)])
    plsc.subcore_barrier()

    def _fetch_idx_start(k, row_v, col_v, sem):
        base = k * C
        r = pltpu.async_copy(ei_hbm.at[0, pl.ds(base, C)], row_v, sem)
        q = pltpu.async_copy(ei_hbm.at[1, pl.ds(base, C)], col_v, sem)
        return (r, q)

    def _fetch_idx_finish(descs, row_v, col_v):
        descs[0].wait()
        descs[1].wait()
        for j in range(C // 16):
            rv = row_v[pl.ds(j * 16, 16)]
            cv = col_v[pl.ds(j * 16, 16)]
            row_v[pl.ds(j * 16, 16)] = jnp.where(rv == cv, DUMMY, rv)

    def _fetch_idx(k, row_v, col_v):
        _fetch_idx_finish(_fetch_idx_start(k, row_v, col_v, gsem0),
                          row_v, col_v)

    rvs = (row_v0, row_v1, row_v2)
    cvs = (col_v0, col_v1, col_v2)
    bufs = (buf0, buf1, buf2)
    isems = (isem0, isem1, isem2)
    gsems = (gsem0, gsem1, gsem2)
    ssems = (ssem0, ssem1, ssem2)

    def _block(chunk_of, n):
        idx_d = [None] * n
        g = [None] * n
        sct = [None] * n

        def idx_start(i):
            idx_d[i] = _fetch_idx_start(chunk_of(i), rvs[i % 3], cvs[i % 3],
                                        isems[i % 3])

        def idx_finish(i):
            _fetch_idx_finish(idx_d[i], rvs[i % 3], cvs[i % 3])

        def gather(i):
            g[i] = pltpu.async_copy(x_hbm.at[cvs[i % 3]], bufs[i % 3],
                                    gsems[i % 3])

        def scatter(i):
            sct[i] = pltpu.async_copy(bufs[i % 3], acc.at[rvs[i % 3]],
                                      ssems[i % 3], add=True)

        idx_start(0)
        idx_start(1)
        idx_start(2)
        idx_finish(0)
        gather(0)
        idx_finish(1)
        g[0].wait()
        scatter(0)
        gather(1)
        idx_finish(2)
        g[1].wait()
        scatter(1)
        gather(2)
        for i in range(3, n):
            sct[i - 3].wait()
            idx_start(i)
            g[i - 1].wait()
            scatter(i - 1)
            idx_finish(i)
            gather(i)
        g[n - 1].wait()
        scatter(n - 1)
        for i in range(n - 3, n):
            sct[i].wait()

    BN = 24

    def _block12(kb, wid):
        _block(lambda i: (kb * BN + i) * NW + wid, BN)
        return wid

    lax.fori_loop(0, FULL // BN, _block12, wid, unroll=False)
    _block(lambda i: ((FULL // BN) * BN + i) * NW + wid, FULL % BN)

    @pl.when(wid < REM)
    def _tail():
        _fetch_idx(FULL * NW + wid, row_v0, col_v0)
        pltpu.async_copy(x_hbm.at[col_v0], buf0, gsem0).wait()
        pltpu.sync_copy(buf0, acc.at[row_v0], add=True)

    plsc.subcore_barrier()

    wds = []
    for b in range(ROWS_PER_TILE // C):
        off = s * ROWS_PER_TILE + b * C
        wds.append(pltpu.async_copy(acc.at[pl.ds(off, C)],
                                    out_hbm.at[c, pl.ds(off, C)], gsem0))
    for wd in wds:
        wd.wait()
    off = s * ROWS_PER_TILE + (ROWS_PER_TILE // C) * C
    pltpu.sync_copy(acc.at[pl.ds(off, rem_rows)],
                    out_hbm.at[c, pl.ds(off, rem_rows)])


_sc_aggregate = functools.partial(
    pl.kernel,
    mesh=plsc.VectorSubcoreMesh(core_axis_name="c", subcore_axis_name="s"),
    out_type=jax.ShapeDtypeStruct((NC, ACC_ROWS, D), jnp.float32),
    scratch_types=[
        pltpu.VMEM((C,), jnp.int32),
        pltpu.VMEM((C,), jnp.int32),
        pltpu.VMEM((C,), jnp.int32),
        pltpu.VMEM((C,), jnp.int32),
        pltpu.VMEM((C,), jnp.int32),
        pltpu.VMEM((C,), jnp.int32),
        pltpu.VMEM((C, D), jnp.float32),
        pltpu.VMEM((C, D), jnp.float32),
        pltpu.VMEM((C, D), jnp.float32),
        pltpu.VMEM_SHARED((ACC_ROWS, D), jnp.float32),
    ] + [pltpu.SemaphoreType.DMA] * 9,
)(_sc_body)


def _mlp_body(x_ref, p_ref, w1_ref, b1_ref, w2_ref, b2_ref, o_ref):
    out = x_ref[...] + p_ref[0] + p_ref[1]
    h = jnp.dot(out, w1_ref[...], preferred_element_type=jnp.float32)
    h = jnp.maximum(h + b1_ref[...], 0.0)
    y = jnp.dot(h, w2_ref[...], preferred_element_type=jnp.float32)
    o_ref[...] = y + b2_ref[...]


MB = 2000


def _mlp(x, partials, W1, b1, W2, b2):
    grid = (N // MB,)
    return pl.pallas_call(
        _mlp_body,
        grid=grid,
        in_specs=[
            pl.BlockSpec((MB, D), lambda i: (i, 0)),
            pl.BlockSpec((NC, MB, D), lambda i: (0, i, 0)),
            pl.BlockSpec((D, D), lambda i: (0, 0)),
            pl.BlockSpec((1, D), lambda i: (0, 0)),
            pl.BlockSpec((D, D), lambda i: (0, 0)),
            pl.BlockSpec((1, D), lambda i: (0, 0)),
        ],
        out_specs=pl.BlockSpec((MB, D), lambda i: (i, 0)),
        out_shape=jax.ShapeDtypeStruct((N, D), jnp.float32),
    )(x, partials, W1, b1.reshape(1, D), W2, b2.reshape(1, D))


def kernel(x, edge_index, W1, b1, W2, b2):
    ei = edge_index.astype(jnp.int32)
    partials = _sc_aggregate(ei, x)
    return _mlp(x, partials, W1, b1, W2, b2)


# speedup vs baseline: 1.0733x; 1.0088x over previous
import functools

import jax
import jax.numpy as jnp
from jax import lax
from jax.experimental import pallas as pl
from jax.experimental.pallas import tpu as pltpu
from jax.experimental.pallas import tpu_sc as plsc

N = 10000
E = 320000
D = 128

NC = 2
NS = 16
NW = NC * NS

C = 128
CHUNKS = E // C
FULL = CHUNKS // NW
REM = CHUNKS % NW
CT = REM * C // NW

ACC_ROWS = 10112
ROWS_PER_TILE = ACC_ROWS // NS
DUMMY = N


def _sc_body(ei_hbm, x_hbm, out_hbm, row_v0, col_v0, row_v1,
             col_v1, row_v2, col_v2, row_t, col_t, buf0, buf1, buf2, acc,
             gsem0, gsem1, gsem2, ssem0, ssem1, ssem2,
             isem0, isem1, isem2):
    c = lax.axis_index("c")
    s = lax.axis_index("s")
    wid = c * NS + s

    def _zero_row(r, carry):
        for j in range(D // 16):
            buf0[r, pl.ds(j * 16, 16)] = jnp.zeros((16,), jnp.float32)
        return carry

    lax.fori_loop(0, C, _zero_row, 0, unroll=False)
    for b in range(ROWS_PER_TILE // C):
        pltpu.sync_copy(buf0, acc.at[pl.ds(s * ROWS_PER_TILE + b * C, C)])
    rem_rows = ROWS_PER_TILE % C
    pltpu.sync_copy(
        buf0.at[pl.ds(0, rem_rows)],
        acc.at[pl.ds(s * ROWS_PER_TILE + (ROWS_PER_TILE // C) * C,
                     rem_rows)])
    plsc.subcore_barrier()

    def _fetch_idx_start(k, row_v, col_v, sem):
        base = k * C
        r = pltpu.async_copy(ei_hbm.at[0, pl.ds(base, C)], row_v, sem)
        q = pltpu.async_copy(ei_hbm.at[1, pl.ds(base, C)], col_v, sem)
        return (r, q)

    def _fetch_idx_finish(descs, row_v, col_v):
        descs[0].wait()
        descs[1].wait()
        for j in range(C // 16):
            rv = row_v[pl.ds(j * 16, 16)]
            cv = col_v[pl.ds(j * 16, 16)]
            row_v[pl.ds(j * 16, 16)] = jnp.where(rv == cv, DUMMY, rv)

    def _fetch_idx(k, row_v, col_v):
        _fetch_idx_finish(_fetch_idx_start(k, row_v, col_v, gsem0),
                          row_v, col_v)

    rvs = (row_v0, row_v1, row_v2)
    cvs = (col_v0, col_v1, col_v2)
    bufs = (buf0, buf1, buf2)
    isems = (isem0, isem1, isem2)
    gsems = (gsem0, gsem1, gsem2)
    ssems = (ssem0, ssem1, ssem2)

    def _block(chunk_of, n):
        idx_d = [None] * n
        g = [None] * n
        sct = [None] * n

        def idx_start(i):
            idx_d[i] = _fetch_idx_start(chunk_of(i), rvs[i % 3], cvs[i % 3],
                                        isems[i % 3])

        def idx_finish(i):
            _fetch_idx_finish(idx_d[i], rvs[i % 3], cvs[i % 3])

        def gather(i):
            g[i] = pltpu.async_copy(x_hbm.at[cvs[i % 3]], bufs[i % 3],
                                    gsems[i % 3])

        def scatter(i):
            sct[i] = pltpu.async_copy(bufs[i % 3], acc.at[rvs[i % 3]],
                                      ssems[i % 3], add=True)

        idx_start(0)
        idx_start(1)
        idx_start(2)
        idx_finish(0)
        gather(0)
        idx_finish(1)
        g[0].wait()
        scatter(0)
        gather(1)
        idx_finish(2)
        g[1].wait()
        scatter(1)
        gather(2)
        for i in range(3, n):
            sct[i - 3].wait()
            idx_start(i)
            g[i - 1].wait()
            scatter(i - 1)
            idx_finish(i)
            gather(i)
        g[n - 1].wait()
        scatter(n - 1)
        for i in range(n - 3, n):
            sct[i].wait()

    BN = 24

    def _block12(kb, wid):
        _block(lambda i: (kb * BN + i) * NW + wid, BN)
        return wid

    lax.fori_loop(0, FULL // BN, _block12, wid, unroll=False)
    _block(lambda i: ((FULL // BN) * BN + i) * NW + wid, FULL % BN)

    tbase = FULL * NW * C + wid * CT
    pltpu.async_copy(ei_hbm.at[0, pl.ds(tbase, CT)], row_t, isem0)
    pltpu.async_copy(ei_hbm.at[1, pl.ds(tbase, CT)], col_t, isem1)
    pltpu.make_async_copy(ei_hbm.at[0, pl.ds(tbase, CT)], row_t,
                          isem0).wait()
    pltpu.make_async_copy(ei_hbm.at[1, pl.ds(tbase, CT)], col_t,
                          isem1).wait()
    rv = row_t[pl.ds(0, 16)]
    cv = col_t[pl.ds(0, 16)]
    row_t[pl.ds(0, 16)] = jnp.where(rv == cv, DUMMY, rv)
    pltpu.async_copy(x_hbm.at[col_t], buf0.at[pl.ds(0, CT)], gsem0).wait()
    pltpu.sync_copy(buf0.at[pl.ds(0, CT)], acc.at[row_t], add=True)

    plsc.subcore_barrier()

    wds = []
    for b in range(ROWS_PER_TILE // C):
        off = s * ROWS_PER_TILE + b * C
        wds.append(pltpu.async_copy(acc.at[pl.ds(off, C)],
                                    out_hbm.at[c, pl.ds(off, C)], gsem0))
    for wd in wds:
        wd.wait()
    off = s * ROWS_PER_TILE + (ROWS_PER_TILE // C) * C
    pltpu.sync_copy(acc.at[pl.ds(off, rem_rows)],
                    out_hbm.at[c, pl.ds(off, rem_rows)])


_sc_aggregate = functools.partial(
    pl.kernel,
    mesh=plsc.VectorSubcoreMesh(core_axis_name="c", subcore_axis_name="s"),
    out_type=jax.ShapeDtypeStruct((NC, ACC_ROWS, D), jnp.float32),
    scratch_types=[
        pltpu.VMEM((C,), jnp.int32),
        pltpu.VMEM((C,), jnp.int32),
        pltpu.VMEM((C,), jnp.int32),
        pltpu.VMEM((C,), jnp.int32),
        pltpu.VMEM((C,), jnp.int32),
        pltpu.VMEM((C,), jnp.int32),
        pltpu.VMEM((CT,), jnp.int32),
        pltpu.VMEM((CT,), jnp.int32),
        pltpu.VMEM((C, D), jnp.float32),
        pltpu.VMEM((C, D), jnp.float32),
        pltpu.VMEM((C, D), jnp.float32),
        pltpu.VMEM_SHARED((ACC_ROWS, D), jnp.float32),
    ] + [pltpu.SemaphoreType.DMA] * 9,
)(_sc_body)


def _mlp_body(x_ref, p_ref, w1_ref, b1_ref, w2_ref, b2_ref, o_ref):
    out = x_ref[...] + p_ref[0] + p_ref[1]
    h = jnp.dot(out, w1_ref[...], preferred_element_type=jnp.float32)
    h = jnp.maximum(h + b1_ref[...], 0.0)
    y = jnp.dot(h, w2_ref[...], preferred_element_type=jnp.float32)
    o_ref[...] = y + b2_ref[...]


MB = 2000


def _mlp(x, partials, W1, b1, W2, b2):
    grid = (N // MB,)
    return pl.pallas_call(
        _mlp_body,
        grid=grid,
        in_specs=[
            pl.BlockSpec((MB, D), lambda i: (i, 0)),
            pl.BlockSpec((NC, MB, D), lambda i: (0, i, 0)),
            pl.BlockSpec((D, D), lambda i: (0, 0)),
            pl.BlockSpec((1, D), lambda i: (0, 0)),
            pl.BlockSpec((D, D), lambda i: (0, 0)),
            pl.BlockSpec((1, D), lambda i: (0, 0)),
        ],
        out_specs=pl.BlockSpec((MB, D), lambda i: (i, 0)),
        out_shape=jax.ShapeDtypeStruct((N, D), jnp.float32),
    )(x, partials, W1, b1.reshape(1, D), W2, b2.reshape(1, D))


def kernel(x, edge_index, W1, b1, W2, b2):
    ei = edge_index.astype(jnp.int32)
    partials = _sc_aggregate(ei, x)
    return _mlp(x, partials, W1, b1, W2, b2)


# speedup vs baseline: 1.0740x; 1.0007x over previous
import functools

import jax
import jax.numpy as jnp
from jax import lax
from jax.experimental import pallas as pl
from jax.experimental.pallas import tpu as pltpu
from jax.experimental.pallas import tpu_sc as plsc

N = 10000
E = 320000
D = 128

NC = 2
NS = 16
NW = NC * NS

C = 128
CHUNKS = E // C
FULL = CHUNKS // NW
REM = CHUNKS % NW
CT = REM * C // NW

ACC_ROWS = 10112
ROWS_PER_TILE = ACC_ROWS // NS
DUMMY = N


def _sc_body(ei_hbm, x_hbm, out_hbm, row_v0, col_v0, row_v1,
             col_v1, row_v2, col_v2, row_t, col_t, buf0, buf1, buf2, acc,
             gsem0, gsem1, gsem2, ssem0, ssem1, ssem2,
             isem0, isem1, isem2):
    c = lax.axis_index("c")
    s = lax.axis_index("s")
    wid = c * NS + s

    def _zero_row(r, carry):
        for j in range(D // 16):
            buf0[r, pl.ds(j * 16, 16)] = jnp.zeros((16,), jnp.float32)
        return carry

    lax.fori_loop(0, C, _zero_row, 0, unroll=False)
    for b in range(ROWS_PER_TILE // C):
        pltpu.sync_copy(buf0, acc.at[pl.ds(s * ROWS_PER_TILE + b * C, C)])
    rem_rows = ROWS_PER_TILE % C
    pltpu.sync_copy(
        buf0.at[pl.ds(0, rem_rows)],
        acc.at[pl.ds(s * ROWS_PER_TILE + (ROWS_PER_TILE // C) * C,
                     rem_rows)])
    plsc.subcore_barrier()

    def _fetch_idx_start(k, row_v, col_v, sem):
        base = k * C
        r = pltpu.async_copy(ei_hbm.at[0, pl.ds(base, C)], row_v, sem)
        q = pltpu.async_copy(ei_hbm.at[1, pl.ds(base, C)], col_v, sem)
        return (r, q)

    def _fetch_idx_finish(descs, row_v, col_v):
        descs[0].wait()
        descs[1].wait()
        for j in range(C // 16):
            rv = row_v[pl.ds(j * 16, 16)]
            cv = col_v[pl.ds(j * 16, 16)]
            row_v[pl.ds(j * 16, 16)] = jnp.where(rv == cv, DUMMY, rv)

    rvs = (row_v0, row_v1, row_v2)
    cvs = (col_v0, col_v1, col_v2)
    bufs = (buf0, buf1, buf2)
    isems = (isem0, isem1, isem2)
    gsems = (gsem0, gsem1, gsem2)
    ssems = (ssem0, ssem1, ssem2)

    def _block(chunk_of, n):
        idx_d = [None] * n
        g = [None] * n
        sct = [None] * n

        def idx_start(i):
            idx_d[i] = _fetch_idx_start(chunk_of(i), rvs[i % 3], cvs[i % 3],
                                        isems[i % 3])

        def idx_finish(i):
            _fetch_idx_finish(idx_d[i], rvs[i % 3], cvs[i % 3])

        def gather(i):
            g[i] = pltpu.async_copy(x_hbm.at[cvs[i % 3]], bufs[i % 3],
                                    gsems[i % 3])

        def scatter(i):
            sct[i] = pltpu.async_copy(bufs[i % 3], acc.at[rvs[i % 3]],
                                      ssems[i % 3], add=True)

        idx_start(0)
        idx_start(1)
        idx_start(2)
        idx_finish(0)
        gather(0)
        idx_finish(1)
        g[0].wait()
        scatter(0)
        gather(1)
        idx_finish(2)
        g[1].wait()
        scatter(1)
        gather(2)
        for i in range(3, n):
            sct[i - 3].wait()
            idx_start(i)
            g[i - 1].wait()
            scatter(i - 1)
            idx_finish(i)
            gather(i)
        g[n - 1].wait()
        scatter(n - 1)
        for i in range(n - 3, n):
            sct[i].wait()

    BN = 24

    def _block_n(kb, wid):
        _block(lambda i: (kb * BN + i) * NW + wid, BN)
        return wid

    lax.fori_loop(0, FULL // BN, _block_n, wid, unroll=False)
    _block(lambda i: ((FULL // BN) * BN + i) * NW + wid, FULL % BN)

    tbase = FULL * NW * C + wid * CT
    pltpu.async_copy(ei_hbm.at[0, pl.ds(tbase, CT)], row_t, isem0)
    pltpu.async_copy(ei_hbm.at[1, pl.ds(tbase, CT)], col_t, isem1)
    pltpu.make_async_copy(ei_hbm.at[0, pl.ds(tbase, CT)], row_t,
                          isem0).wait()
    pltpu.make_async_copy(ei_hbm.at[1, pl.ds(tbase, CT)], col_t,
                          isem1).wait()
    rv = row_t[pl.ds(0, 16)]
    cv = col_t[pl.ds(0, 16)]
    row_t[pl.ds(0, 16)] = jnp.where(rv == cv, DUMMY, rv)
    pltpu.async_copy(x_hbm.at[col_t], buf0.at[pl.ds(0, CT)], gsem0).wait()
    pltpu.sync_copy(buf0.at[pl.ds(0, CT)], acc.at[row_t], add=True)

    plsc.subcore_barrier()

    wds = []
    for b in range(ROWS_PER_TILE // C):
        off = s * ROWS_PER_TILE + b * C
        wds.append(pltpu.async_copy(acc.at[pl.ds(off, C)],
                                    out_hbm.at[c, pl.ds(off, C)], gsem0))
    for wd in wds:
        wd.wait()
    off = s * ROWS_PER_TILE + (ROWS_PER_TILE // C) * C
    pltpu.sync_copy(acc.at[pl.ds(off, rem_rows)],
                    out_hbm.at[c, pl.ds(off, rem_rows)])


_sc_aggregate = functools.partial(
    pl.kernel,
    mesh=plsc.VectorSubcoreMesh(core_axis_name="c", subcore_axis_name="s"),
    out_type=jax.ShapeDtypeStruct((NC, ACC_ROWS, D), jnp.float32),
    scratch_types=[
        pltpu.VMEM((C,), jnp.int32),
        pltpu.VMEM((C,), jnp.int32),
        pltpu.VMEM((C,), jnp.int32),
        pltpu.VMEM((C,), jnp.int32),
        pltpu.VMEM((C,), jnp.int32),
        pltpu.VMEM((C,), jnp.int32),
        pltpu.VMEM((CT,), jnp.int32),
        pltpu.VMEM((CT,), jnp.int32),
        pltpu.VMEM((C, D), jnp.float32),
        pltpu.VMEM((C, D), jnp.float32),
        pltpu.VMEM((C, D), jnp.float32),
        pltpu.VMEM_SHARED((ACC_ROWS, D), jnp.float32),
    ] + [pltpu.SemaphoreType.DMA] * 9,
)(_sc_body)


def _mlp_body(x_ref, p_ref, w1_ref, b1_ref, w2_ref, b2_ref, o_ref):
    out = x_ref[...] + p_ref[0] + p_ref[1]
    h = jnp.dot(out, w1_ref[...], preferred_element_type=jnp.float32)
    h = jnp.maximum(h + b1_ref[...], 0.0)
    y = jnp.dot(h, w2_ref[...], preferred_element_type=jnp.float32)
    o_ref[...] = y + b2_ref[...]


MB = 2000


def _mlp(x, partials, W1, b1, W2, b2):
    grid = (N // MB,)
    return pl.pallas_call(
        _mlp_body,
        grid=grid,
        in_specs=[
            pl.BlockSpec((MB, D), lambda i: (i, 0)),
            pl.BlockSpec((NC, MB, D), lambda i: (0, i, 0)),
            pl.BlockSpec((D, D), lambda i: (0, 0)),
            pl.BlockSpec((1, D), lambda i: (0, 0)),
            pl.BlockSpec((D, D), lambda i: (0, 0)),
            pl.BlockSpec((1, D), lambda i: (0, 0)),
        ],
        out_specs=pl.BlockSpec((MB, D), lambda i: (i, 0)),
        out_shape=jax.ShapeDtypeStruct((N, D), jnp.float32),
    )(x, partials, W1, b1.reshape(1, D), W2, b2.reshape(1, D))


def kernel(x, edge_index, W1, b1, W2, b2):
    ei = edge_index.astype(jnp.int32)
    partials = _sc_aggregate(ei, x)
    return _mlp(x, partials, W1, b1, W2, b2)
